# Initial kernel scaffold; baseline (speedup 1.0000x reference)
import jax
import jax.numpy as jnp
from jax.experimental import pallas as pl


def kernel(tokens, positions, token_table, position_table):
    raise NotImplementedError("write your pallas kernel here")


# speedup vs baseline: 1.0780x; 1.0780x over previous
import functools
import jax
import jax.numpy as jnp
from jax import lax
from jax.experimental import pallas as pl
from jax.experimental.pallas import tpu as pltpu
from jax.experimental.pallas import tpu_sc as plsc

D = 768
LANES = 16
NV = D // LANES
NC, NS = 2, 16
NW = NC * NS
CB = 32


@functools.lru_cache(maxsize=None)
def _emb_kernel(nrows: int, plen: int):
    RPW = nrows // NW
    NCH = RPW // CB
    assert nrows % NW == 0 and RPW % CB == 0

    mesh = plsc.VectorSubcoreMesh(core_axis_name="c", subcore_axis_name="s")

    @functools.partial(
        pl.kernel,
        mesh=mesh,
        compiler_params=pltpu.CompilerParams(needs_layout_passes=False),
        out_type=jax.ShapeDtypeStruct((nrows, D), jnp.float32),
        scratch_types=[
            pltpu.VMEM((RPW,), jnp.int32),
            pltpu.VMEM((RPW,), jnp.int32),
            pltpu.VMEM((plen * D,), jnp.float32),
            pltpu.VMEM((CB, D), jnp.float32),
            pltpu.SemaphoreType.DMA,
        ],
    )
    def k(tok_hbm, poff_hbm, table_hbm, ptab_hbm, out_hbm,
          tok_v, poff_v, ptab_v, buf_v, gsem):
        wid = lax.axis_index("s") * NC + lax.axis_index("c")
        base = wid * RPW
        pltpu.sync_copy(tok_hbm.at[pl.ds(base, RPW)], tok_v)
        pltpu.sync_copy(poff_hbm.at[pl.ds(base, RPW)], poff_v)
        pltpu.sync_copy(ptab_hbm, ptab_v)

        col0 = lax.iota(jnp.int32, LANES)

        def chunk(c, carry):
            rbase = c * CB
            idx = tok_v.at[pl.ds(rbase, CB)]
            pltpu.async_copy(table_hbm.at[idx], buf_v, gsem).wait()

            def row(r, carry2):
                poff = plsc.load_gather(
                    poff_v, [jnp.full((LANES,), rbase + r, jnp.int32)])
                rsplat = jnp.full((LANES,), r, jnp.int32)
                for j in range(NV):
                    col = j * LANES + col0
                    pv = plsc.load_gather(ptab_v, [poff + col])
                    plsc.addupdate_scatter(buf_v, [rsplat, col], pv)
                return carry2

            lax.fori_loop(0, CB, row, 0)
            pltpu.sync_copy(buf_v, out_hbm.at[pl.ds(base + rbase, CB)])
            return carry

        lax.fori_loop(0, NCH, chunk, 0)

    return k


def kernel(tokens, positions, token_table, position_table):
    B, T = tokens.shape
    nrows = B * T
    tok = tokens.reshape(nrows).astype(jnp.int32)
    poff = (positions.reshape(nrows) * D).astype(jnp.int32)
    ptab = position_table.reshape(-1)
    out = _emb_kernel(nrows, position_table.shape[0])(
        tok, poff, token_table, ptab)
    return out.reshape(B, T, D)


# speedup vs baseline: 1.2039x; 1.1168x over previous
import functools
import jax
import jax.numpy as jnp
from jax import lax
from jax.experimental import pallas as pl
from jax.experimental.pallas import tpu as pltpu
from jax.experimental.pallas import tpu_sc as plsc

D = 768
LANES = 16
NV = D // LANES
NC, NS = 2, 16
NW = NC * NS
CB = 32


@functools.lru_cache(maxsize=None)
def _emb_kernel(nrows: int, plen: int):
    RPW = nrows // NW
    NCH = RPW // CB
    assert nrows % NW == 0 and RPW % CB == 0 and NCH % 2 == 1 and NCH >= 5

    mesh = plsc.VectorSubcoreMesh(core_axis_name="c", subcore_axis_name="s")

    @functools.partial(
        pl.kernel,
        mesh=mesh,
        compiler_params=pltpu.CompilerParams(needs_layout_passes=False),
        out_type=jax.ShapeDtypeStruct((nrows, D), jnp.float32),
        scratch_types=[
            pltpu.VMEM((RPW,), jnp.int32),
            pltpu.VMEM((RPW,), jnp.int32),
            pltpu.VMEM((plen * D,), jnp.float32),
            pltpu.VMEM((CB, D), jnp.float32),
            pltpu.VMEM((CB, D), jnp.float32),
            pltpu.SemaphoreType.DMA,
            pltpu.SemaphoreType.DMA,
            pltpu.SemaphoreType.DMA,
            pltpu.SemaphoreType.DMA,
        ],
    )
    def k(tok_hbm, poff_hbm, table_hbm, ptab_hbm, out_hbm,
          tok_v, poff_v, ptab_v, buf0, buf1, gsem0, gsem1, ssem0, ssem1):
        wid = lax.axis_index("s") * NC + lax.axis_index("c")
        base = wid * RPW
        pltpu.sync_copy(tok_hbm.at[pl.ds(base, RPW)], tok_v)
        pltpu.sync_copy(poff_hbm.at[pl.ds(base, RPW)], poff_v)
        pltpu.sync_copy(ptab_hbm, ptab_v)

        col0 = lax.iota(jnp.int32, LANES)

        def gather_start(c, buf, sem):
            pltpu.async_copy(table_hbm.at[tok_v.at[pl.ds(c * CB, CB)]], buf, sem)

        def gather_wait(c, buf, sem):
            pltpu.make_async_copy(
                table_hbm.at[tok_v.at[pl.ds(c * CB, CB)]], buf, sem).wait()

        def scatter_start(c, buf, sem):
            pltpu.async_copy(buf, out_hbm.at[pl.ds(base + c * CB, CB)], sem)

        def scatter_wait(c, buf, sem):
            pltpu.make_async_copy(
                buf, out_hbm.at[pl.ds(base + c * CB, CB)], sem).wait()

        def add_pos(c, buf):
            rbase = c * CB

            def row(r, carry):
                poff = plsc.load_gather(
                    poff_v, [jnp.full((LANES,), rbase + r, jnp.int32)])
                rsplat = jnp.full((LANES,), r, jnp.int32)
                for j in range(NV):
                    col = j * LANES + col0
                    pv = plsc.load_gather(ptab_v, [poff + col])
                    plsc.addupdate_scatter(buf, [rsplat, col], pv)
                return carry

            lax.fori_loop(0, CB, row, 0)

        gather_start(0, buf0, gsem0)
        gather_start(1, buf1, gsem1)
        gather_wait(0, buf0, gsem0)
        add_pos(0, buf0)
        scatter_start(0, buf0, ssem0)

        def body(i, carry):
            c0 = 1 + 2 * i
            c1 = c0 + 1
            scatter_wait(c0 - 1, buf0, ssem0)
            gather_start(c0 + 1, buf0, gsem0)
            gather_wait(c0, buf1, gsem1)
            add_pos(c0, buf1)
            scatter_start(c0, buf1, ssem1)

            scatter_wait(c0, buf1, ssem1)
            gather_start(c1 + 1, buf1, gsem1)
            gather_wait(c1, buf0, gsem0)
            add_pos(c1, buf0)
            scatter_start(c1, buf0, ssem0)
            return carry

        lax.fori_loop(0, (NCH - 3) // 2, body, 0)

        cA = NCH - 2
        cB = NCH - 1
        scatter_wait(cA - 1, buf0, ssem0)
        gather_start(cA + 1, buf0, gsem0)
        gather_wait(cA, buf1, gsem1)
        add_pos(cA, buf1)
        scatter_start(cA, buf1, ssem1)

        scatter_wait(cA, buf1, ssem1)
        gather_wait(cB, buf0, gsem0)
        add_pos(cB, buf0)
        scatter_start(cB, buf0, ssem0)
        scatter_wait(cB, buf0, ssem0)

    return k


def kernel(tokens, positions, token_table, position_table):
    B, T = tokens.shape
    nrows = B * T
    tok = tokens.reshape(nrows).astype(jnp.int32)
    poff = (positions.reshape(nrows) * D).astype(jnp.int32)
    ptab = position_table.reshape(-1)
    out = _emb_kernel(nrows, position_table.shape[0])(
        tok, poff, token_table, ptab)
    return out.reshape(B, T, D)


# speedup vs baseline: 1.8633x; 1.5478x over previous
import functools
import jax
import jax.numpy as jnp
from jax import lax
from jax.experimental import pallas as pl
from jax.experimental.pallas import tpu as pltpu
from jax.experimental.pallas import tpu_sc as plsc

D = 768
LANES = 16
NV = D // LANES
NC, NS = 2, 16
NW = NC * NS
CB = 32


@functools.lru_cache(maxsize=None)
def _emb_kernel(nrows: int, plen: int):
    RPW = nrows // NW
    NCH = RPW // CB
    assert nrows % NW == 0 and RPW % CB == 0 and NCH % 2 == 1 and NCH >= 5

    mesh = plsc.VectorSubcoreMesh(core_axis_name="c", subcore_axis_name="s")

    @functools.partial(
        pl.kernel,
        mesh=mesh,
        compiler_params=pltpu.CompilerParams(
            needs_layout_passes=False, use_tc_tiling_on_sc=True),
        out_type=jax.ShapeDtypeStruct((nrows, D), jnp.float32),
        scratch_types=[
            pltpu.VMEM((RPW,), jnp.int32),
            pltpu.VMEM((RPW,), jnp.int32),
            pltpu.VMEM((plen * D,), jnp.float32),
            pltpu.VMEM((CB, D), jnp.float32),
            pltpu.VMEM((CB, D), jnp.float32),
            pltpu.SemaphoreType.DMA,
            pltpu.SemaphoreType.DMA,
            pltpu.SemaphoreType.DMA,
            pltpu.SemaphoreType.DMA,
        ],
    )
    def k(tok_hbm, poff_hbm, table_hbm, ptab_hbm, out_hbm,
          tok_v, poff_v, ptab_v, buf0, buf1, gsem0, gsem1, ssem0, ssem1):
        wid = lax.axis_index("s") * NC + lax.axis_index("c")
        base = wid * RPW
        pltpu.sync_copy(tok_hbm.at[pl.ds(base, RPW)], tok_v)
        pltpu.sync_copy(poff_hbm.at[pl.ds(base, RPW)], poff_v)
        pltpu.sync_copy(ptab_hbm, ptab_v)

        col0 = lax.iota(jnp.int32, LANES)

        def gather_start(c, buf, sem):
            pltpu.async_copy(table_hbm.at[tok_v.at[pl.ds(c * CB, CB)]], buf, sem)

        def gather_wait(c, buf, sem):
            pltpu.make_async_copy(
                table_hbm.at[tok_v.at[pl.ds(c * CB, CB)]], buf, sem).wait()

        def scatter_start(c, buf, sem):
            pltpu.async_copy(buf, out_hbm.at[pl.ds(base + c * CB, CB)], sem)

        def scatter_wait(c, buf, sem):
            pltpu.make_async_copy(
                buf, out_hbm.at[pl.ds(base + c * CB, CB)], sem).wait()

        def add_pos(c, buf):
            rbase = c * CB

            def row(r, carry):
                poff = plsc.load_gather(
                    poff_v, [jnp.full((LANES,), rbase + r, jnp.int32)])
                rsplat = jnp.full((LANES,), r, jnp.int32)
                for j in range(NV):
                    col = j * LANES + col0
                    pv = plsc.load_gather(ptab_v, [poff + col])
                    plsc.addupdate_scatter(buf, [rsplat, col], pv)
                return carry

            lax.fori_loop(0, CB, row, 0)

        gather_start(0, buf0, gsem0)
        gather_start(1, buf1, gsem1)
        gather_wait(0, buf0, gsem0)
        add_pos(0, buf0)
        scatter_start(0, buf0, ssem0)

        def body(i, carry):
            c0 = 1 + 2 * i
            c1 = c0 + 1
            scatter_wait(c0 - 1, buf0, ssem0)
            gather_start(c0 + 1, buf0, gsem0)
            gather_wait(c0, buf1, gsem1)
            add_pos(c0, buf1)
            scatter_start(c0, buf1, ssem1)

            scatter_wait(c0, buf1, ssem1)
            gather_start(c1 + 1, buf1, gsem1)
            gather_wait(c1, buf0, gsem0)
            add_pos(c1, buf0)
            scatter_start(c1, buf0, ssem0)
            return carry

        lax.fori_loop(0, (NCH - 3) // 2, body, 0)

        cA = NCH - 2
        cB = NCH - 1
        scatter_wait(cA - 1, buf0, ssem0)
        gather_start(cA + 1, buf0, gsem0)
        gather_wait(cA, buf1, gsem1)
        add_pos(cA, buf1)
        scatter_start(cA, buf1, ssem1)

        scatter_wait(cA, buf1, ssem1)
        gather_wait(cB, buf0, gsem0)
        add_pos(cB, buf0)
        scatter_start(cB, buf0, ssem0)
        scatter_wait(cB, buf0, ssem0)

    return k


def kernel(tokens, positions, token_table, position_table):
    B, T = tokens.shape
    nrows = B * T
    tok = tokens.T.reshape(nrows).astype(jnp.int32)
    poff = (positions.T.reshape(nrows) * D).astype(jnp.int32)
    ptab = position_table.reshape(-1)
    out = _emb_kernel(nrows, position_table.shape[0])(
        tok, poff, token_table, ptab)
    return out.reshape(T, B, D).transpose(1, 0, 2)


# speedup vs baseline: 2.0152x; 1.0815x over previous
import functools
import jax
import jax.numpy as jnp
from jax import lax
from jax.experimental import pallas as pl
from jax.experimental.pallas import tpu as pltpu
from jax.experimental.pallas import tpu_sc as plsc

D = 768
LANES = 16
NV = D // LANES
NC, NS = 2, 16
NW = NC * NS
CB = 16
UJ = 8
NSLOT = 4


@functools.lru_cache(maxsize=None)
def _emb_kernel(nrows: int, plen: int):
    RPW = nrows // NW
    NCH = RPW // CB
    NB = (NCH - 2) // NSLOT
    assert nrows % NW == 0 and RPW % CB == 0 and NCH == NB * NSLOT + 2

    mesh = plsc.VectorSubcoreMesh(core_axis_name="c", subcore_axis_name="s")

    @functools.partial(
        pl.kernel,
        mesh=mesh,
        compiler_params=pltpu.CompilerParams(
            needs_layout_passes=False, use_tc_tiling_on_sc=True),
        out_type=jax.ShapeDtypeStruct((nrows, D), jnp.float32),
        scratch_types=[
            pltpu.VMEM((RPW,), jnp.int32),
            pltpu.VMEM((RPW,), jnp.int32),
            pltpu.VMEM((plen * D,), jnp.float32),
            pltpu.VMEM((CB, D), jnp.float32),
            pltpu.VMEM((CB, D), jnp.float32),
            pltpu.VMEM((CB, D), jnp.float32),
            pltpu.VMEM((CB, D), jnp.float32),
            pltpu.SemaphoreType.DMA,
            pltpu.SemaphoreType.DMA,
            pltpu.SemaphoreType.DMA,
            pltpu.SemaphoreType.DMA,
            pltpu.SemaphoreType.DMA,
            pltpu.SemaphoreType.DMA,
            pltpu.SemaphoreType.DMA,
            pltpu.SemaphoreType.DMA,
        ],
    )
    def k(tok_hbm, poff_hbm, table_hbm, ptab_hbm, out_hbm,
          tok_v, poff_v, ptab_v, b0, b1, b2, b3,
          g0, g1, g2, g3, s0, s1, s2, s3):
        bufs = (b0, b1, b2, b3)
        gsems = (g0, g1, g2, g3)
        ssems = (s0, s1, s2, s3)
        wid = lax.axis_index("s") * NC + lax.axis_index("c")
        base = wid * RPW
        pltpu.sync_copy(tok_hbm.at[pl.ds(base, RPW)], tok_v)
        pltpu.sync_copy(poff_hbm.at[pl.ds(base, RPW)], poff_v)
        pltpu.sync_copy(ptab_hbm, ptab_v)

        def gather_start(c, sl):
            pltpu.async_copy(
                table_hbm.at[tok_v.at[pl.ds(c * CB, CB)]], bufs[sl], gsems[sl])

        def gather_wait(c, sl):
            pltpu.make_async_copy(
                table_hbm.at[tok_v.at[pl.ds(c * CB, CB)]], bufs[sl],
                gsems[sl]).wait()

        def scatter_start(c, sl):
            pltpu.async_copy(
                bufs[sl], out_hbm.at[pl.ds(base + c * CB, CB)], ssems[sl])

        def scatter_wait(c, sl):
            pltpu.make_async_copy(
                bufs[sl], out_hbm.at[pl.ds(base + c * CB, CB)],
                ssems[sl]).wait()

        def add_pos(c, buf):
            pvec = poff_v[pl.ds(c * CB, CB)]
            poffs = [pvec[r] for r in range(CB)]

            def jblk(jj, carry):
                cb = jj * (UJ * LANES)
                for u in range(UJ):
                    col = u * LANES
                    for r in range(CB):
                        pv = ptab_v[pl.ds(poffs[r] + cb + col, LANES)]
                        plsc.addupdate(buf.at[r, pl.ds(cb + col, LANES)], pv)
                return carry

            lax.fori_loop(0, NV // UJ, jblk, 0)

        for sl in range(NSLOT):
            gather_start(sl, sl)

        def body(i, carry):
            for sl in range(NSLOT):
                c = i * NSLOT + sl
                gather_wait(c, sl)

                @pl.when((c > 0) & (c + (NSLOT - 1) < NCH))
                def _():
                    scatter_wait(c - 1, (sl + NSLOT - 1) % NSLOT)
                    gather_start(c + (NSLOT - 1), (sl + NSLOT - 1) % NSLOT)

                add_pos(c, bufs[sl])
                scatter_start(c, sl)
            return carry

        lax.fori_loop(0, NB, body, 0)

        cA = NCH - 2
        gather_wait(cA, 0)
        add_pos(cA, bufs[0])
        scatter_start(cA, 0)
        cB = NCH - 1
        gather_wait(cB, 1)
        add_pos(cB, bufs[1])
        scatter_start(cB, 1)
        scatter_wait(NCH - 4, 2)
        scatter_wait(NCH - 3, 3)
        scatter_wait(cA, 0)
        scatter_wait(cB, 1)

    return k


def kernel(tokens, positions, token_table, position_table):
    B, T = tokens.shape
    nrows = B * T
    tok = tokens.T.reshape(nrows).astype(jnp.int32)
    poff = (positions.T.reshape(nrows) * D).astype(jnp.int32)
    ptab = position_table.reshape(-1)
    out = _emb_kernel(nrows, position_table.shape[0])(
        tok, poff, token_table, ptab)
    return out.reshape(T, B, D).transpose(1, 0, 2)


# speedup vs baseline: 4.6474x; 2.3061x over previous
import functools
import jax
import jax.numpy as jnp
from jax import lax
from jax.experimental import pallas as pl
from jax.experimental.pallas import tpu as pltpu
from jax.experimental.pallas import tpu_sc as plsc

D = 768
LANES = 16
NV = D // LANES
NC, NS = 2, 16
NW = NC * NS
CB = 16
UJ = 4
NSLOT = 4


@functools.lru_cache(maxsize=None)
def _emb_kernel(nrows: int, plen: int):
    RPW = nrows // NW
    NCH = RPW // CB
    NB = (NCH - 2) // NSLOT
    assert nrows % NW == 0 and RPW % CB == 0 and NCH == NB * NSLOT + 2

    mesh = plsc.VectorSubcoreMesh(core_axis_name="c", subcore_axis_name="s")

    @functools.partial(
        pl.kernel,
        mesh=mesh,
        compiler_params=pltpu.CompilerParams(
            needs_layout_passes=False, use_tc_tiling_on_sc=True),
        out_type=jax.ShapeDtypeStruct((nrows, D), jnp.float32),
        scratch_types=[
            pltpu.VMEM((RPW,), jnp.int32),
            pltpu.VMEM((RPW,), jnp.int32),
            pltpu.VMEM((plen * D,), jnp.float32),
            pltpu.VMEM((CB, D), jnp.float32),
            pltpu.VMEM((CB, D), jnp.float32),
            pltpu.VMEM((CB, D), jnp.float32),
            pltpu.VMEM((CB, D), jnp.float32),
            pltpu.SemaphoreType.DMA,
            pltpu.SemaphoreType.DMA,
            pltpu.SemaphoreType.DMA,
            pltpu.SemaphoreType.DMA,
            pltpu.SemaphoreType.DMA,
            pltpu.SemaphoreType.DMA,
            pltpu.SemaphoreType.DMA,
            pltpu.SemaphoreType.DMA,
        ],
    )
    def k(tok_hbm, poff_hbm, table_hbm, ptab_hbm, out_hbm,
          tok_v, poff_v, ptab_v, b0, b1, b2, b3,
          g0, g1, g2, g3, s0, s1, s2, s3):
        bufs = (b0, b1, b2, b3)
        gsems = (g0, g1, g2, g3)
        ssems = (s0, s1, s2, s3)
        wid = lax.axis_index("s") * NC + lax.axis_index("c")
        base = wid * RPW
        pltpu.sync_copy(tok_hbm.at[pl.ds(base, RPW)], tok_v)
        pltpu.sync_copy(poff_hbm.at[pl.ds(base, RPW)], poff_v)
        pltpu.sync_copy(ptab_hbm, ptab_v)

        def gather_start(c, sl):
            pltpu.async_copy(
                table_hbm.at[tok_v.at[pl.ds(c * CB, CB)]], bufs[sl], gsems[sl])

        def gather_wait(c, sl):
            pltpu.make_async_copy(
                table_hbm.at[tok_v.at[pl.ds(c * CB, CB)]], bufs[sl],
                gsems[sl]).wait()

        def scatter_start(c, sl):
            pltpu.async_copy(
                bufs[sl], out_hbm.at[pl.ds(base + c * CB, CB)], ssems[sl])

        def scatter_wait(c, sl):
            pltpu.make_async_copy(
                bufs[sl], out_hbm.at[pl.ds(base + c * CB, CB)],
                ssems[sl]).wait()

        def add_pos(c, buf):
            pvec = poff_v[pl.ds(c * CB, CB)]
            poffs = [pvec[r] for r in range(CB)]

            @plsc.parallel_loop(0, NV, unroll=UJ)
            def jblk(jj):
                col = jj * LANES
                for r in range(CB):
                    pv = ptab_v[pl.ds(poffs[r] + col, LANES)]
                    plsc.addupdate(buf.at[r, pl.ds(col, LANES)], pv)

        for sl in range(NSLOT):
            gather_start(sl, sl)

        def body(i, carry):
            for sl in range(NSLOT):
                c = i * NSLOT + sl
                gather_wait(c, sl)

                @pl.when((c > 0) & (c + (NSLOT - 1) < NCH))
                def _():
                    scatter_wait(c - 1, (sl + NSLOT - 1) % NSLOT)
                    gather_start(c + (NSLOT - 1), (sl + NSLOT - 1) % NSLOT)

                add_pos(c, bufs[sl])
                scatter_start(c, sl)
            return carry

        lax.fori_loop(0, NB, body, 0)

        cA = NCH - 2
        gather_wait(cA, 0)
        add_pos(cA, bufs[0])
        scatter_start(cA, 0)
        cB = NCH - 1
        gather_wait(cB, 1)
        add_pos(cB, bufs[1])
        scatter_start(cB, 1)
        scatter_wait(NCH - 4, 2)
        scatter_wait(NCH - 3, 3)
        scatter_wait(cA, 0)
        scatter_wait(cB, 1)

    return k


def kernel(tokens, positions, token_table, position_table):
    B, T = tokens.shape
    nrows = B * T
    tok = tokens.T.reshape(nrows).astype(jnp.int32)
    poff = (positions.T.reshape(nrows) * D).astype(jnp.int32)
    ptab = position_table.reshape(-1)
    out = _emb_kernel(nrows, position_table.shape[0])(
        tok, poff, token_table, ptab)
    return out.reshape(T, B, D).transpose(1, 0, 2)


# speedup vs baseline: 4.6774x; 1.0064x over previous
import functools
import jax
import jax.numpy as jnp
from jax import lax
from jax.experimental import pallas as pl
from jax.experimental.pallas import tpu as pltpu
from jax.experimental.pallas import tpu_sc as plsc

D = 768
LANES = 16
NV = D // LANES
NC, NS = 2, 16
NW = NC * NS
CB = 16
UJ = 4
NSLOT = 4


@functools.lru_cache(maxsize=None)
def _emb_kernel(nrows: int, plen: int):
    RPW = nrows // NW
    NCH = RPW // CB
    NB = (NCH - 2) // NSLOT
    assert nrows % NW == 0 and RPW % CB == 0 and NCH == NB * NSLOT + 2

    mesh = plsc.VectorSubcoreMesh(core_axis_name="c", subcore_axis_name="s")

    @functools.partial(
        pl.kernel,
        mesh=mesh,
        compiler_params=pltpu.CompilerParams(
            needs_layout_passes=False, use_tc_tiling_on_sc=True,
            disable_bounds_checks=True),
        out_type=jax.ShapeDtypeStruct((nrows, D), jnp.float32),
        scratch_types=[
            pltpu.VMEM((RPW,), jnp.int32),
            pltpu.VMEM((RPW,), jnp.int32),
            pltpu.VMEM((plen * D,), jnp.float32),
            pltpu.VMEM((CB, D), jnp.float32),
            pltpu.VMEM((CB, D), jnp.float32),
            pltpu.VMEM((CB, D), jnp.float32),
            pltpu.VMEM((CB, D), jnp.float32),
            pltpu.SemaphoreType.DMA,
            pltpu.SemaphoreType.DMA,
            pltpu.SemaphoreType.DMA,
            pltpu.SemaphoreType.DMA,
            pltpu.SemaphoreType.DMA,
            pltpu.SemaphoreType.DMA,
            pltpu.SemaphoreType.DMA,
            pltpu.SemaphoreType.DMA,
        ],
    )
    def k(tok_hbm, poff_hbm, table_hbm, ptab_hbm, out_hbm,
          tok_v, poff_v, ptab_v, b0, b1, b2, b3,
          g0, g1, g2, g3, s0, s1, s2, s3):
        bufs = (b0, b1, b2, b3)
        gsems = (g0, g1, g2, g3)
        ssems = (s0, s1, s2, s3)
        wid = lax.axis_index("s") * NC + lax.axis_index("c")
        base = wid * RPW
        pltpu.sync_copy(tok_hbm.at[pl.ds(base, RPW)], tok_v)

        def gather_start(c, sl):
            pltpu.async_copy(
                table_hbm.at[tok_v.at[pl.ds(c * CB, CB)]], bufs[sl], gsems[sl])

        def gather_wait(c, sl):
            pltpu.make_async_copy(
                table_hbm.at[tok_v.at[pl.ds(c * CB, CB)]], bufs[sl],
                gsems[sl]).wait()

        def scatter_start(c, sl):
            pltpu.async_copy(
                bufs[sl], out_hbm.at[pl.ds(base + c * CB, CB)], ssems[sl])

        def scatter_wait(c, sl):
            pltpu.make_async_copy(
                bufs[sl], out_hbm.at[pl.ds(base + c * CB, CB)],
                ssems[sl]).wait()

        def add_pos(c, buf):
            pvec = poff_v[pl.ds(c * CB, CB)]
            poffs = [pvec[r] for r in range(CB)]

            @plsc.parallel_loop(0, NV, unroll=UJ)
            def jblk(jj):
                col = jj * LANES
                for r in range(CB):
                    pv = ptab_v[pl.ds(poffs[r] + col, LANES)]
                    plsc.addupdate(buf.at[r, pl.ds(col, LANES)], pv)

        for sl in range(NSLOT):
            gather_start(sl, sl)
        pltpu.sync_copy(poff_hbm.at[pl.ds(base, RPW)], poff_v)
        pltpu.sync_copy(ptab_hbm, ptab_v)

        def body(i, carry):
            for sl in range(NSLOT):
                c = i * NSLOT + sl
                gather_wait(c, sl)

                @pl.when((c > 0) & (c + (NSLOT - 1) < NCH))
                def _():
                    scatter_wait(c - 1, (sl + NSLOT - 1) % NSLOT)
                    gather_start(c + (NSLOT - 1), (sl + NSLOT - 1) % NSLOT)

                add_pos(c, bufs[sl])
                scatter_start(c, sl)
            return carry

        lax.fori_loop(0, NB, body, 0)

        cA = NCH - 2
        gather_wait(cA, 0)
        add_pos(cA, bufs[0])
        scatter_start(cA, 0)
        cB = NCH - 1
        gather_wait(cB, 1)
        add_pos(cB, bufs[1])
        scatter_start(cB, 1)
        scatter_wait(NCH - 4, 2)
        scatter_wait(NCH - 3, 3)
        scatter_wait(cA, 0)
        scatter_wait(cB, 1)

    return k


def kernel(tokens, positions, token_table, position_table):
    B, T = tokens.shape
    nrows = B * T
    tok = tokens.T.reshape(nrows).astype(jnp.int32)
    poff = (positions.T.reshape(nrows) * D).astype(jnp.int32)
    ptab = position_table.reshape(-1)
    out = _emb_kernel(nrows, position_table.shape[0])(
        tok, poff, token_table, ptab)
    return out.reshape(T, B, D).transpose(1, 0, 2)
